# Initial kernel scaffold; baseline (speedup 1.0000x reference)
import jax
import jax.numpy as jnp
from jax.experimental import pallas as pl


def kernel(parent_coord, parent_offset, s_coord, s_offset, s_feat):
    raise NotImplementedError("write your pallas kernel here")


# speedup vs baseline: 2.0564x; 2.0564x over previous
import functools

import jax
import jax.numpy as jnp
from jax.experimental import pallas as pl
from jax.experimental.pallas import tpu as pltpu

_N = 16384
_M = 4096
_D = 64
_K = 3
_BLK = 256


def _knn_block_kernel(p_ref, sx_ref, sf_ref, out_ref):
    p = p_ref[...]
    sx = sx_ref[...]
    nn2 = (p[:, 0:1] * p[:, 0:1]
           + p[:, 1:2] * p[:, 1:2]
           + p[:, 2:3] * p[:, 2:3])
    mm2 = (sx[0:1, :] * sx[0:1, :]
           + sx[1:2, :] * sx[1:2, :]
           + sx[2:3, :] * sx[2:3, :])
    pb = p.astype(jnp.bfloat16).astype(jnp.float32)
    sb = sx.astype(jnp.bfloat16).astype(jnp.float32)
    dot = (pb[:, 0:1] * sb[0:1, :]
           + pb[:, 1:2] * sb[1:2, :]
           + pb[:, 2:3] * sb[2:3, :])
    d2 = jnp.maximum(nn2 + mm2 - 2.0 * dot, 0.0)

    iota = jax.lax.broadcasted_iota(jnp.int32, (_BLK, _M), 1)
    d = d2
    w_hot = None
    recips = []
    hots = []
    for k in range(_K):
        v = jnp.min(d, axis=1, keepdims=True)
        eq = d == v
        idx = jnp.min(jnp.where(eq, iota, _M), axis=1, keepdims=True)
        hot = jnp.logical_and(eq, iota == idx)
        dist = jnp.sqrt(v + 1e-12)
        recips.append(1.0 / (dist + 1e-8))
        hots.append(hot)
        if k < _K - 1:
            d = jnp.where(hot, jnp.inf, d)
    norm = recips[0] + recips[1] + recips[2]
    w_hot = (jnp.where(hots[0], recips[0], 0.0)
             + jnp.where(hots[1], recips[1], 0.0)
             + jnp.where(hots[2], recips[2], 0.0)) / norm
    out_ref[...] = jax.lax.dot(
        w_hot, sf_ref[...],
        precision=jax.lax.Precision.HIGHEST,
        preferred_element_type=jnp.float32)


@jax.jit
def _feat_propagation(parent_coord, s_coord, s_feat):
    p_pad = jnp.zeros((_N, 128), jnp.float32).at[:, :3].set(parent_coord)
    sx = jnp.zeros((8, _M), jnp.float32).at[:3, :].set(s_coord.T)
    grid = (_N // _BLK,)
    return pl.pallas_call(
        _knn_block_kernel,
        grid=grid,
        in_specs=[
            pl.BlockSpec((_BLK, 128), lambda i: (i, 0)),
            pl.BlockSpec((8, _M), lambda i: (0, 0)),
            pl.BlockSpec((_M, _D), lambda i: (0, 0)),
        ],
        out_specs=pl.BlockSpec((_BLK, _D), lambda i: (i, 0)),
        out_shape=jax.ShapeDtypeStruct((_N, _D), jnp.float32),
        compiler_params=pltpu.CompilerParams(
            dimension_semantics=("parallel",)),
    )(p_pad, sx, s_feat)


def kernel(parent_coord, parent_offset, s_coord, s_offset, s_feat):
    del parent_offset, s_offset
    return _feat_propagation(parent_coord, s_coord, s_feat)


# speedup vs baseline: 2.2891x; 1.1132x over previous
import jax
import jax.numpy as jnp
from jax.experimental import pallas as pl
from jax.experimental.pallas import tpu as pltpu

_N = 16384
_M = 4096
_D = 64
_K = 3
_BLK = 256


def _knn_block_kernel(p_ref, sx_ref, sf_ref, out_ref):
    p = p_ref[...]
    sx = sx_ref[...]
    nn2 = (p[:, 0:1] * p[:, 0:1]
           + p[:, 1:2] * p[:, 1:2]
           + p[:, 2:3] * p[:, 2:3])
    mm2 = (sx[0:1, :] * sx[0:1, :]
           + sx[1:2, :] * sx[1:2, :]
           + sx[2:3, :] * sx[2:3, :])
    dot = jax.lax.dot(p.astype(jnp.bfloat16), sx.astype(jnp.bfloat16),
                      preferred_element_type=jnp.float32)
    d2 = jnp.maximum(nn2 + mm2 - 2.0 * dot, 0.0)

    iota = jax.lax.broadcasted_iota(jnp.int32, (_BLK, _M), 1)
    d = d2
    recips = []
    hots = []
    for k in range(_K):
        v = jnp.min(d, axis=1, keepdims=True)
        eq = d == v
        idx = jnp.min(jnp.where(eq, iota, _M), axis=1, keepdims=True)
        hot = iota == idx
        dist = jnp.sqrt(v + 1e-12)
        recips.append(1.0 / (dist + 1e-8))
        hots.append(hot)
        if k < _K - 1:
            d = jnp.where(hot, jnp.inf, d)
    norm = recips[0] + recips[1] + recips[2]
    w0 = recips[0] / norm
    w1 = recips[1] / norm
    w2 = recips[2] / norm
    w_hot = jnp.where(hots[0], w0,
                      jnp.where(hots[1], w1,
                                jnp.where(hots[2], w2, 0.0)))
    out_ref[...] = jax.lax.dot(
        w_hot, sf_ref[...],
        precision=jax.lax.Precision.HIGHEST,
        preferred_element_type=jnp.float32)


@jax.jit
def _feat_propagation(parent_coord, s_coord, s_feat):
    p_pad = jnp.zeros((_N, 128), jnp.float32).at[:, :3].set(parent_coord)
    sx = jnp.zeros((128, _M), jnp.float32).at[:3, :].set(s_coord.T)
    grid = (_N // _BLK,)
    return pl.pallas_call(
        _knn_block_kernel,
        grid=grid,
        in_specs=[
            pl.BlockSpec((_BLK, 128), lambda i: (i, 0)),
            pl.BlockSpec((128, _M), lambda i: (0, 0)),
            pl.BlockSpec((_M, _D), lambda i: (0, 0)),
        ],
        out_specs=pl.BlockSpec((_BLK, _D), lambda i: (i, 0)),
        out_shape=jax.ShapeDtypeStruct((_N, _D), jnp.float32),
        compiler_params=pltpu.CompilerParams(
            dimension_semantics=("parallel",)),
    )(p_pad, sx, s_feat)


def kernel(parent_coord, parent_offset, s_coord, s_offset, s_feat):
    del parent_offset, s_offset
    return _feat_propagation(parent_coord, s_coord, s_feat)


# speedup vs baseline: 3.4899x; 1.5245x over previous
import jax
import jax.numpy as jnp
from jax.experimental import pallas as pl
from jax.experimental.pallas import tpu as pltpu

_N = 16384
_M = 4096
_D = 64
_K = 3
_BLK = 256


def _knn_block_kernel(p_ref, sx_ref, sfh_ref, sfl_ref, out_ref):
    p = p_ref[...]
    sx = sx_ref[...]
    nn2 = (p[:, 0:1] * p[:, 0:1]
           + p[:, 1:2] * p[:, 1:2]
           + p[:, 2:3] * p[:, 2:3])
    mm2 = (sx[0:1, :] * sx[0:1, :]
           + sx[1:2, :] * sx[1:2, :]
           + sx[2:3, :] * sx[2:3, :])
    dot = jax.lax.dot(p.astype(jnp.bfloat16), sx.astype(jnp.bfloat16),
                      preferred_element_type=jnp.float32)
    d2 = jnp.maximum(nn2 + mm2 - 2.0 * dot, 0.0)

    iota = jax.lax.broadcasted_iota(jnp.int32, (_BLK, _M), 1)
    d = d2
    recips = []
    hots = []
    for k in range(_K):
        v = jnp.min(d, axis=1, keepdims=True)
        eq = d == v
        idx = jnp.min(jnp.where(eq, iota, _M), axis=1, keepdims=True)
        hot = iota == idx
        dist = jnp.sqrt(v + 1e-12)
        recips.append(1.0 / (dist + 1e-8))
        hots.append(hot)
        if k < _K - 1:
            d = jnp.where(hot, jnp.inf, d)
    norm = recips[0] + recips[1] + recips[2]
    w0 = recips[0] / norm
    w1 = recips[1] / norm
    w2 = recips[2] / norm
    w_hot = jnp.where(hots[0], w0,
                      jnp.where(hots[1], w1,
                                jnp.where(hots[2], w2, 0.0)))
    w_bf = w_hot.astype(jnp.bfloat16)
    out_ref[...] = (
        jax.lax.dot(w_bf, sfh_ref[...], preferred_element_type=jnp.float32)
        + jax.lax.dot(w_bf, sfl_ref[...], preferred_element_type=jnp.float32))


@jax.jit
def _feat_propagation(parent_coord, s_coord, s_feat):
    p_pad = jnp.zeros((_N, 128), jnp.float32).at[:, :3].set(parent_coord)
    sx = jnp.zeros((128, _M), jnp.float32).at[:3, :].set(s_coord.T)
    sf_hi = s_feat.astype(jnp.bfloat16)
    sf_lo = (s_feat - sf_hi.astype(jnp.float32)).astype(jnp.bfloat16)
    grid = (_N // _BLK,)
    return pl.pallas_call(
        _knn_block_kernel,
        grid=grid,
        in_specs=[
            pl.BlockSpec((_BLK, 128), lambda i: (i, 0)),
            pl.BlockSpec((128, _M), lambda i: (0, 0)),
            pl.BlockSpec((_M, _D), lambda i: (0, 0)),
            pl.BlockSpec((_M, _D), lambda i: (0, 0)),
        ],
        out_specs=pl.BlockSpec((_BLK, _D), lambda i: (i, 0)),
        out_shape=jax.ShapeDtypeStruct((_N, _D), jnp.float32),
        compiler_params=pltpu.CompilerParams(
            dimension_semantics=("parallel",)),
    )(p_pad, sx, sf_hi, sf_lo)


def kernel(parent_coord, parent_offset, s_coord, s_offset, s_feat):
    del parent_offset, s_offset
    return _feat_propagation(parent_coord, s_coord, s_feat)
